# baseline (speedup 1.0000x reference)
import functools

import jax
import jax.numpy as jnp
from jax import lax
from jax.experimental import pallas as pl
from jax.experimental.pallas import tpu as pltpu
from jax.experimental.pallas import tpu_sc as plsc

H = 16
U = 729
N = 38416
L = 16
NW = 32
STEP = 1200
CHUNK = 1216


_mesh = plsc.VectorSubcoreMesh(core_axis_name="c", subcore_axis_name="s")


@functools.partial(
    pl.kernel,
    mesh=_mesh,
    compiler_params=pltpu.CompilerParams(
        needs_layout_passes=False, use_tc_tiling_on_sc=False
    ),
    out_type=jax.ShapeDtypeStruct((H * N,), jnp.float32),
    scratch_types=[
        pltpu.VMEM((H, U), jnp.float32),
        pltpu.VMEM((CHUNK,), jnp.int32),
        pltpu.VMEM((H, CHUNK), jnp.float32),
        pltpu.SemaphoreType.DMA,
        pltpu.SemaphoreType.DMA,
        pltpu.SemaphoreType.DMA,
    ],
)
def _gather_bias(bias_hbm, idx_hbm, out_hbm, bias_v, idx_v, out_v,
                 sem_b, sem_i, sem_o):
    cid = lax.axis_index("c")
    sid = lax.axis_index("s")
    wid = sid * 2 + cid
    base = wid * STEP

    cp_b = pltpu.async_copy(bias_hbm, bias_v, sem_b)
    cp_i = pltpu.async_copy(idx_hbm.at[pl.ds(base, CHUNK)], idx_v, sem_i)
    cp_i.wait()
    cp_b.wait()

    hvs = [jnp.full((L,), h, jnp.int32) for h in range(H)]

    HALF = CHUNK // 2
    out_cps = []
    for half in range(2):
        off = half * HALF

        @plsc.parallel_loop(off, off + HALF, L, unroll=2)
        def _body(s):
            iv = idx_v[pl.ds(s, L)]
            for h in range(H):
                out_v[h, pl.ds(s, L)] = plsc.load_gather(bias_v, [hvs[h], iv])

        for h in range(H):
            out_cps.append(
                pltpu.async_copy(out_v.at[h, pl.ds(off, HALF)],
                                 out_hbm.at[pl.ds(h * N + base + off, HALF)],
                                 sem_o))
    for cp in out_cps:
        cp.wait()


def kernel(bias, index):
    out = _gather_bias(bias, index)
    return out.reshape(H, 196, 196)


# speedup vs baseline: 1.0346x; 1.0346x over previous
import functools

import jax
import jax.numpy as jnp
from jax import lax
from jax.experimental import pallas as pl
from jax.experimental.pallas import tpu as pltpu
from jax.experimental.pallas import tpu_sc as plsc

H = 16
U = 729
N = 38416
L = 16
NW = 32
STEP = 1200
CHUNK = 1216


_mesh = plsc.VectorSubcoreMesh(core_axis_name="c", subcore_axis_name="s")


@functools.partial(
    pl.kernel,
    mesh=_mesh,
    compiler_params=pltpu.CompilerParams(
        needs_layout_passes=False, use_tc_tiling_on_sc=False
    ),
    out_type=jax.ShapeDtypeStruct((H, N), jnp.float32),
    scratch_types=[
        pltpu.VMEM((H, U), jnp.float32),
        pltpu.VMEM((CHUNK,), jnp.int32),
        pltpu.VMEM((H, CHUNK), jnp.float32),
        pltpu.SemaphoreType.DMA,
        pltpu.SemaphoreType.DMA,
        pltpu.SemaphoreType.DMA,
    ],
)
def _gather_bias(bias_hbm, idx_hbm, out_hbm, bias_v, idx_v, out_v,
                 sem_b, sem_i, sem_o):
    cid = lax.axis_index("c")
    sid = lax.axis_index("s")
    wid = sid * 2 + cid
    base = wid * STEP

    cp_b = pltpu.async_copy(bias_hbm, bias_v, sem_b)
    cp_i = pltpu.async_copy(idx_hbm.at[pl.ds(base, CHUNK)], idx_v, sem_i)
    cp_i.wait()
    cp_b.wait()

    hvs = [jnp.full((L,), h, jnp.int32) for h in range(H)]

    @plsc.parallel_loop(0, CHUNK, L, unroll=2)
    def _body(s):
        iv = idx_v[pl.ds(s, L)]
        for h in range(H):
            out_v[h, pl.ds(s, L)] = plsc.load_gather(bias_v, [hvs[h], iv])

    pltpu.sync_copy(out_v, out_hbm.at[:, pl.ds(base, CHUNK)])


def kernel(bias, index):
    out = _gather_bias(bias, index)
    return out.reshape(H, 196, 196)
